# Initial kernel scaffold; baseline (speedup 1.0000x reference)
import jax
import jax.numpy as jnp
from jax import lax
from jax.experimental import pallas as pl
from jax.experimental.pallas import tpu as pltpu
from jax.experimental.pallas import tpu_sc as plsc

VOCAB = 400000
DIM = 300
BATCH = 4096
SEQ = 50

NUM_WORKERS = 32
SEGS = 2 * BATCH
SEG_PER_W = SEGS // NUM_WORKERS
LANES = 16
NCHUNK = 19
OUT_BLOCK = 16


def _sc_kernel(table_hbm, idx_hbm, out_hbm,
               idx_v, rows0, rows1, ob0, ob1,
               gsem0, gsem1, osem0, osem1):
    wid = lax.axis_index("c") * 16 + lax.axis_index("s")
    base = wid * SEG_PER_W

    pltpu.sync_copy(idx_hbm.at[pl.ds(base, SEG_PER_W)], idx_v)

    iota = lax.broadcasted_iota(jnp.int32, (LANES,), 0)
    col_bases = [min(16 * c, DIM - LANES) for c in range(NCHUNK)]

    rows_bufs = (rows0, rows1)
    gsems = (gsem0, gsem1)
    out_bufs = (ob0, ob1)
    osems = (osem0, osem1)

    def issue_gather(seg, buf, sem):
        pltpu.async_copy(table_hbm.at[idx_v.at[seg]], buf, sem)

    issue_gather(0, rows0, gsem0)
    issue_gather(1, rows1, gsem1)

    def reduce_rows(buf):
        def body(r, accs):
            row_idx = jnp.full((LANES,), r, jnp.int32)
            return tuple(
                accs[c] + plsc.load_gather(buf, [row_idx, iota + col_bases[c]])
                for c in range(NCHUNK))
        zeros = tuple(jnp.zeros((LANES,), jnp.float32) for _ in range(NCHUNK))
        return lax.fori_loop(0, SEQ, body, zeros)

    scale = jnp.float32(1.0 / SEQ)

    @pl.loop(0, SEG_PER_W, step=2)
    def _(s0):
        for b in range(2):
            seg = s0 + b
            buf, sem = rows_bufs[b], gsems[b]
            pltpu.make_async_copy(table_hbm.at[idx_v.at[seg]], buf, sem).wait()
            accs = reduce_rows(buf)

            @pl.when(seg + 2 < SEG_PER_W)
            def _():
                issue_gather(seg + 2, buf, sem)

            grp = (seg // OUT_BLOCK) % 2

            @pl.when(jnp.logical_and(seg % OUT_BLOCK == 0, seg >= 2 * OUT_BLOCK))
            def _():
                for g in range(2):
                    @pl.when(grp == g)
                    def _(g=g):
                        pltpu.make_async_copy(
                            out_bufs[g],
                            out_hbm.at[pl.ds(0, OUT_BLOCK)],
                            osems[g]).wait()

            row_in_blk = jnp.full((LANES,), seg % OUT_BLOCK, jnp.int32)
            for g in range(2):
                @pl.when(grp == g)
                def _(g=g):
                    for c in range(NCHUNK):
                        plsc.store_scatter(out_bufs[g],
                                           [row_in_blk, iota + col_bases[c]],
                                           accs[c] * scale)

            @pl.when(seg % OUT_BLOCK == OUT_BLOCK - 1)
            def _():
                blk0 = seg - (OUT_BLOCK - 1)
                for g in range(2):
                    @pl.when(grp == g)
                    def _(g=g):
                        pltpu.async_copy(
                            out_bufs[g],
                            out_hbm.at[pl.ds(base + blk0, OUT_BLOCK)],
                            osems[g])

    pltpu.make_async_copy(ob0, out_hbm.at[pl.ds(0, OUT_BLOCK)], osem0).wait()
    pltpu.make_async_copy(ob1, out_hbm.at[pl.ds(0, OUT_BLOCK)], osem1).wait()


@jax.jit
def kernel(premises, hypothesis, glove_embeddings):
    idx = jnp.concatenate([premises, hypothesis], axis=0)

    mesh = plsc.VectorSubcoreMesh(core_axis_name="c", subcore_axis_name="s")
    run = pl.kernel(
        _sc_kernel,
        out_type=jax.ShapeDtypeStruct((SEGS, DIM), jnp.float32),
        mesh=mesh,
        scratch_types=[
            pltpu.VMEM((SEG_PER_W, SEQ), jnp.int32),
            pltpu.VMEM((SEQ, DIM), jnp.float32),
            pltpu.VMEM((SEQ, DIM), jnp.float32),
            pltpu.VMEM((OUT_BLOCK, DIM), jnp.float32),
            pltpu.VMEM((OUT_BLOCK, DIM), jnp.float32),
            pltpu.SemaphoreType.DMA,
            pltpu.SemaphoreType.DMA,
            pltpu.SemaphoreType.DMA,
            pltpu.SemaphoreType.DMA,
        ],
    )
    out = run(glove_embeddings, idx)
    return out[:BATCH], out[BATCH:]


# speedup vs baseline: 1.0034x; 1.0034x over previous
import dataclasses

import jax
import jax.numpy as jnp
from jax import lax
from jax.experimental import pallas as pl
from jax.experimental.pallas import tpu as pltpu
from jax.experimental.pallas import tpu_sc as plsc

VOCAB = 400000
DIM = 300
PDIM = 304
BATCH = 4096
SEQ = 50

NUM_WORKERS = 32
SEGS = 2 * BATCH
SEG_PER_W = SEGS // NUM_WORKERS
LANES = 16
NCHUNK = PDIM // LANES
OUT_BLOCK = 16


def _sc_kernel(table_hbm, idx_hbm, out_hbm,
               idx_v, rows0, rows1, ob0, ob1,
               gsem0, gsem1, osem0, osem1):
    wid = lax.axis_index("c") * 16 + lax.axis_index("s")
    base = pl.multiple_of(wid * SEG_PER_W, SEG_PER_W)

    pltpu.sync_copy(idx_hbm.at[pl.ds(base, SEG_PER_W)], idx_v)

    rows_bufs = (rows0, rows1)
    gsems = (gsem0, gsem1)
    out_bufs = (ob0, ob1)
    osems = (osem0, osem1)

    def issue_gather(seg, buf, sem):
        pltpu.async_copy(table_hbm.at[idx_v.at[seg]], buf, sem)

    issue_gather(0, rows0, gsem0)
    issue_gather(1, rows1, gsem1)

    def reduce_rows(buf):
        def body(r, accs):
            return tuple(
                accs[c] + buf[r, pl.ds(16 * c, LANES)]
                for c in range(NCHUNK))
        zeros = tuple(jnp.zeros((LANES,), jnp.float32) for _ in range(NCHUNK))
        return lax.fori_loop(0, SEQ, body, zeros)

    scale = jnp.float32(1.0 / SEQ)

    @pl.loop(0, SEG_PER_W, step=2)
    def _(s0):
        for b in range(2):
            seg = s0 + b
            buf, sem = rows_bufs[b], gsems[b]
            pltpu.make_async_copy(table_hbm.at[idx_v.at[seg]], buf, sem).wait()
            accs = reduce_rows(buf)

            @pl.when(seg + 2 < SEG_PER_W)
            def _():
                issue_gather(seg + 2, buf, sem)

            grp = (seg // OUT_BLOCK) % 2

            @pl.when(jnp.logical_and(seg % OUT_BLOCK == 0,
                                     seg >= 2 * OUT_BLOCK))
            def _():
                for g in range(2):
                    @pl.when(grp == g)
                    def _(g=g):
                        pltpu.make_async_copy(
                            out_bufs[g],
                            out_hbm.at[pl.ds(0, OUT_BLOCK)],
                            osems[g]).wait()

            row = seg % OUT_BLOCK
            for g in range(2):
                @pl.when(grp == g)
                def _(g=g):
                    for c in range(NCHUNK):
                        out_bufs[g][row, pl.ds(16 * c, LANES)] = (
                            accs[c] * scale)

            @pl.when(seg % OUT_BLOCK == OUT_BLOCK - 1)
            def _():
                blk0 = pl.multiple_of(seg - (OUT_BLOCK - 1), OUT_BLOCK)
                for g in range(2):
                    @pl.when(grp == g)
                    def _(g=g, blk0=blk0):
                        pltpu.async_copy(
                            out_bufs[g],
                            out_hbm.at[pl.ds(pl.multiple_of(base + blk0,
                                                            OUT_BLOCK),
                                             OUT_BLOCK)],
                            osems[g])

    pltpu.make_async_copy(ob0, out_hbm.at[pl.ds(0, OUT_BLOCK)], osem0).wait()
    pltpu.make_async_copy(ob1, out_hbm.at[pl.ds(0, OUT_BLOCK)], osem1).wait()


@jax.jit
def kernel(premises, hypothesis, glove_embeddings):
    idx = jnp.concatenate([premises, hypothesis], axis=0)
    table = jnp.pad(glove_embeddings, ((0, 0), (0, PDIM - DIM)))

    mesh = plsc.VectorSubcoreMesh(core_axis_name="c", subcore_axis_name="s")
    cp = pltpu.CompilerParams()
    for fld, val in (("needs_layout_passes", False),
                     ("use_tc_tiling_on_sc", False)):
        if fld in pltpu.CompilerParams.__dataclass_fields__:
            cp = dataclasses.replace(cp, **{fld: val})
    run = pl.kernel(
        _sc_kernel,
        out_type=jax.ShapeDtypeStruct((SEGS, PDIM), jnp.float32),
        mesh=mesh,
        compiler_params=cp,
        scratch_types=[
            pltpu.VMEM((SEG_PER_W, SEQ), jnp.int32),
            pltpu.VMEM((SEQ, PDIM), jnp.float32),
            pltpu.VMEM((SEQ, PDIM), jnp.float32),
            pltpu.VMEM((OUT_BLOCK, PDIM), jnp.float32),
            pltpu.VMEM((OUT_BLOCK, PDIM), jnp.float32),
            pltpu.SemaphoreType.DMA,
            pltpu.SemaphoreType.DMA,
            pltpu.SemaphoreType.DMA,
            pltpu.SemaphoreType.DMA,
        ],
    )
    out = run(table, idx)
    return out[:BATCH, :DIM], out[BATCH:, :DIM]


# speedup vs baseline: 1.6437x; 1.6381x over previous
import dataclasses

import jax
import jax.numpy as jnp
from jax import lax
from jax.experimental import pallas as pl
from jax.experimental.pallas import tpu as pltpu
from jax.experimental.pallas import tpu_sc as plsc

VOCAB = 400000
DIM = 300
BATCH = 4096
SEQ = 50

NUM_WORKERS = 32
SEGS = 2 * BATCH
SEG_PER_W = SEGS // NUM_WORKERS
LANES = 16
ODIM = 304
OUT_BLOCK = 16
NCHUNK = 19


def _sc_kernel(table_hbm, tail_hbm, idx_hbm, out_hbm,
               idx_v, m0a, m1a, ta, m0b, m1b, tb, ob0, ob1,
               gsem0, gsem1, osem0, osem1):
    wid = lax.axis_index("c") * 16 + lax.axis_index("s")
    base = pl.multiple_of(wid * SEG_PER_W, SEG_PER_W)

    pltpu.sync_copy(idx_hbm.at[pl.ds(base, SEG_PER_W)], idx_v)

    bufs = ((m0a, m1a, ta), (m0b, m1b, tb))
    gsems = (gsem0, gsem1)
    out_bufs = (ob0, ob1)
    osems = (osem0, osem1)

    def issue_gather(seg, b3, sem):
        m0, m1, t = b3
        pltpu.async_copy(table_hbm.at[idx_v.at[seg], pl.ds(0, 128)], m0, sem)
        pltpu.async_copy(table_hbm.at[idx_v.at[seg], pl.ds(128, 128)], m1, sem)
        pltpu.async_copy(tail_hbm.at[idx_v.at[seg]], t, sem)

    def wait_gather(seg, b3, sem):
        m0, m1, t = b3
        pltpu.make_async_copy(table_hbm.at[idx_v.at[seg], pl.ds(0, 128)],
                              m0, sem).wait()
        pltpu.make_async_copy(table_hbm.at[idx_v.at[seg], pl.ds(128, 128)],
                              m1, sem).wait()
        pltpu.make_async_copy(tail_hbm.at[idx_v.at[seg]], t, sem).wait()

    issue_gather(0, bufs[0], gsem0)
    issue_gather(1, bufs[1], gsem1)

    def reduce_rows(b3):
        m0, m1, t = b3

        def body(r, accs):
            new = [accs[c] + m0[r, pl.ds(16 * c, LANES)] for c in range(8)]
            new += [accs[8 + c] + m1[r, pl.ds(16 * c, LANES)]
                    for c in range(8)]
            new += [accs[16 + c] + t[r, pl.ds(16 * c, LANES)]
                    for c in range(3)]
            return tuple(new)

        zeros = tuple(jnp.zeros((LANES,), jnp.float32) for _ in range(NCHUNK))
        return lax.fori_loop(0, SEQ, body, zeros)

    scale = jnp.float32(1.0 / SEQ)

    @pl.loop(0, SEG_PER_W, step=2)
    def _(s0):
        for b in range(2):
            seg = s0 + b
            wait_gather(seg, bufs[b], gsems[b])
            accs = reduce_rows(bufs[b])

            @pl.when(seg + 2 < SEG_PER_W)
            def _():
                issue_gather(seg + 2, bufs[b], gsems[b])

            grp = (seg // OUT_BLOCK) % 2

            @pl.when(jnp.logical_and(seg % OUT_BLOCK == 0,
                                     seg >= 2 * OUT_BLOCK))
            def _():
                for g in range(2):
                    @pl.when(grp == g)
                    def _(g=g):
                        pltpu.make_async_copy(
                            out_bufs[g],
                            out_hbm.at[pl.ds(0, OUT_BLOCK)],
                            osems[g]).wait()

            row = seg % OUT_BLOCK
            for g in range(2):
                @pl.when(grp == g)
                def _(g=g):
                    for c in range(NCHUNK):
                        out_bufs[g][row, pl.ds(16 * c, LANES)] = (
                            accs[c] * scale)

            @pl.when(seg % OUT_BLOCK == OUT_BLOCK - 1)
            def _():
                blk0 = pl.multiple_of(seg - (OUT_BLOCK - 1), OUT_BLOCK)
                for g in range(2):
                    @pl.when(grp == g)
                    def _(g=g, blk0=blk0):
                        pltpu.async_copy(
                            out_bufs[g],
                            out_hbm.at[pl.ds(pl.multiple_of(base + blk0,
                                                            OUT_BLOCK),
                                             OUT_BLOCK)],
                            osems[g])

    pltpu.make_async_copy(ob0, out_hbm.at[pl.ds(0, OUT_BLOCK)], osem0).wait()
    pltpu.make_async_copy(ob1, out_hbm.at[pl.ds(0, OUT_BLOCK)], osem1).wait()


@jax.jit
def kernel(premises, hypothesis, glove_embeddings):
    idx = jnp.concatenate([premises, hypothesis], axis=0)
    tail = jnp.pad(glove_embeddings[:, 256:DIM],
                   ((0, 0), (0, 128 - (DIM - 256))))

    mesh = plsc.VectorSubcoreMesh(core_axis_name="c", subcore_axis_name="s")
    cp = pltpu.CompilerParams()
    if "needs_layout_passes" in pltpu.CompilerParams.__dataclass_fields__:
        cp = dataclasses.replace(cp, needs_layout_passes=False)
    run = pl.kernel(
        _sc_kernel,
        out_type=jax.ShapeDtypeStruct((SEGS, ODIM), jnp.float32),
        mesh=mesh,
        compiler_params=cp,
        scratch_types=[
            pltpu.VMEM((SEG_PER_W, SEQ), jnp.int32),
            pltpu.VMEM((SEQ, 128), jnp.float32),
            pltpu.VMEM((SEQ, 128), jnp.float32),
            pltpu.VMEM((SEQ, 128), jnp.float32),
            pltpu.VMEM((SEQ, 128), jnp.float32),
            pltpu.VMEM((SEQ, 128), jnp.float32),
            pltpu.VMEM((SEQ, 128), jnp.float32),
            pltpu.VMEM((OUT_BLOCK, ODIM), jnp.float32),
            pltpu.VMEM((OUT_BLOCK, ODIM), jnp.float32),
            pltpu.SemaphoreType.DMA,
            pltpu.SemaphoreType.DMA,
            pltpu.SemaphoreType.DMA,
            pltpu.SemaphoreType.DMA,
        ],
    )
    out = run(glove_embeddings, tail, idx)
    return out[:BATCH, :DIM], out[BATCH:, :DIM]
